# baseline (speedup 1.0000x reference)
import functools

import jax
import jax.numpy as jnp
from jax import lax
from jax.experimental import pallas as pl
from jax.experimental.pallas import tpu as pltpu
from jax.experimental.pallas import tpu_sc as plsc

N_FINE = 100000
N_COARSE = 50000
IN_CH = 256
SKIP_CH = 128
OUT_CH = 256

NUM_CORES = 2
NUM_SUBCORES = 16
NUM_WORKERS = NUM_CORES * NUM_SUBCORES

CHUNK = 80
HALF = N_FINE // 2
SLOTS_H = 20
HALF_PAD = SLOTS_H * NUM_WORKERS * CHUNK


def _sc_gather_body(idx_hbm, down_hbm, out_hbm, idx_all,
                    rows0, rows1, rows2, rows3,
                    sg0, sg1, sg2, sg3, sw0, sw1, sw2, sw3):
    wid = lax.axis_index("s") * NUM_CORES + lax.axis_index("c")
    rows = (rows0, rows1, rows2, rows3)
    sg = (sg0, sg1, sg2, sg3)
    sw = (sw0, sw1, sw2, sw3)

    def gather(i, s):
        return pltpu.make_async_copy(down_hbm.at[idx_all.at[i]], rows[s], sg[s])

    def writeback(i, s):
        c = wid + i * NUM_WORKERS
        dst = out_hbm.at[pl.ds(c * CHUNK, CHUNK)]
        return pltpu.make_async_copy(rows[s], dst, sw[s])

    pltpu.sync_copy(idx_hbm.at[:, wid], idx_all)
    gather(0, 0).start()
    gather(1, 1).start()

    def step(t, carry):
        for b in range(4):
            i = 4 * t + b
            s = b
            gather(i, s).wait()
            writeback(i, s).start()

            @pl.when(i + 2 < SLOTS_H)
            def _():
                @pl.when(i >= 2)
                def _():
                    writeback(i - 2, (b + 2) % 4).wait()

                gather(i + 2, (b + 2) % 4).start()

        return carry

    lax.fori_loop(0, SLOTS_H // 4, step, 0)

    for s in range(4):
        writeback(SLOTS_H - 4 + s, s).wait()


_sc_gather_half = pl.kernel(
    _sc_gather_body,
    out_type=jax.ShapeDtypeStruct((HALF_PAD, IN_CH), jnp.float32),
    mesh=plsc.VectorSubcoreMesh(core_axis_name="c", subcore_axis_name="s"),
    scratch_types=[
        pltpu.VMEM((SLOTS_H, CHUNK), jnp.int32),
        pltpu.VMEM((CHUNK, IN_CH), jnp.float32),
        pltpu.VMEM((CHUNK, IN_CH), jnp.float32),
        pltpu.VMEM((CHUNK, IN_CH), jnp.float32),
        pltpu.VMEM((CHUNK, IN_CH), jnp.float32),
        pltpu.SemaphoreType.DMA,
        pltpu.SemaphoreType.DMA,
        pltpu.SemaphoreType.DMA,
        pltpu.SemaphoreType.DMA,
        pltpu.SemaphoreType.DMA,
        pltpu.SemaphoreType.DMA,
        pltpu.SemaphoreType.DMA,
        pltpu.SemaphoreType.DMA,
    ],
)


def _tc_fused_body(res_ref, gat_ref, wskip_ref, wproj_ref, bias_ref, out_ref):
    out_ref[...] = (
        jnp.dot(res_ref[...], wskip_ref[...], preferred_element_type=jnp.float32)
        + jnp.dot(gat_ref[...], wproj_ref[...], preferred_element_type=jnp.float32)
        + bias_ref[...]
    )


def _tc_fused_body2(res_ref, gat_ref, wskip_ref, wproj_ref, bias_ref, part_ref,
                    out_ref):
    del part_ref
    _tc_fused_body(res_ref, gat_ref, wskip_ref, wproj_ref, bias_ref, out_ref)


ROWS_BLK = 5000
GRID_H = HALF // ROWS_BLK

_W_SPECS = [
    pl.BlockSpec((SKIP_CH, OUT_CH), lambda i: (0, 0)),
    pl.BlockSpec((IN_CH, OUT_CH), lambda i: (0, 0)),
    pl.BlockSpec((1, OUT_CH), lambda i: (0, 0)),
]

_tc_first = pl.pallas_call(
    _tc_fused_body,
    grid=(GRID_H,),
    in_specs=[
        pl.BlockSpec((ROWS_BLK, SKIP_CH), lambda i: (i, 0)),
        pl.BlockSpec((ROWS_BLK, IN_CH), lambda i: (i, 0)),
        *_W_SPECS,
    ],
    out_specs=pl.BlockSpec((ROWS_BLK, OUT_CH), lambda i: (i, 0)),
    out_shape=jax.ShapeDtypeStruct((N_FINE, OUT_CH), jnp.float32),
)

_tc_second = pl.pallas_call(
    _tc_fused_body2,
    grid=(GRID_H,),
    in_specs=[
        pl.BlockSpec((ROWS_BLK, SKIP_CH), lambda i: (i + GRID_H, 0)),
        pl.BlockSpec((ROWS_BLK, IN_CH), lambda i: (i, 0)),
        *_W_SPECS,
        pl.BlockSpec(memory_space=pl.ANY),
    ],
    out_specs=pl.BlockSpec((ROWS_BLK, OUT_CH), lambda i: (i + GRID_H, 0)),
    out_shape=jax.ShapeDtypeStruct((N_FINE, OUT_CH), jnp.float32),
    input_output_aliases={5: 0},
)


def _half_idx(buffers_half):
    pad = SLOTS_H * NUM_WORKERS * CHUNK - HALF
    return jnp.pad(buffers_half, (0, pad)).reshape(SLOTS_H, NUM_WORKERS, CHUNK)


def kernel(residual, down, buffers, W_proj, b_proj, W_skip, b_skip):
    bias = (b_proj + b_skip).reshape(1, OUT_CH)
    g0 = _sc_gather_half(_half_idx(buffers[:HALF]), down)
    g1 = _sc_gather_half(_half_idx(buffers[HALF:]), down)
    part = _tc_first(residual, g0, W_skip, W_proj, bias)
    return _tc_second(residual, g1, W_skip, W_proj, bias, part)


# speedup vs baseline: 1.5229x; 1.5229x over previous
import functools

import jax
import jax.numpy as jnp
from jax import lax
from jax.experimental import pallas as pl
from jax.experimental.pallas import tpu as pltpu
from jax.experimental.pallas import tpu_sc as plsc

N_FINE = 100000
N_COARSE = 50000
IN_CH = 256
SKIP_CH = 128
OUT_CH = 256

NUM_CORES = 2
NUM_SUBCORES = 16
NUM_WORKERS = NUM_CORES * NUM_SUBCORES

CHUNK = 80
HALF = N_FINE // 2
N_CHUNKS_H = HALF // CHUNK
SLOTS_H = 20


def _sc_gather_body(idx_hbm, down_hbm, out_hbm, idx_all, rows0, rows1,
                    sem_g0, sem_g1, sem_w0, sem_w1):
    wid = lax.axis_index("s") * NUM_CORES + lax.axis_index("c")

    def gather(i, rows, sem):
        return pltpu.make_async_copy(down_hbm.at[idx_all.at[i]], rows, sem)

    def writeback(i, rows, sem):
        c = wid + i * NUM_WORKERS
        return pltpu.make_async_copy(rows, out_hbm.at[pl.ds(c * CHUNK, CHUNK)], sem)

    def valid(i):
        return wid + i * NUM_WORKERS < N_CHUNKS_H

    pltpu.sync_copy(idx_hbm.at[:, wid], idx_all)
    gather(0, rows0, sem_g0).start()

    def step(t, carry):
        i = 2 * t
        gather(i, rows0, sem_g0).wait()
        writeback(i, rows0, sem_w0).start()

        @pl.when(valid(i + 1))
        def _():
            @pl.when(t > 0)
            def _():
                writeback(i - 1, rows1, sem_w1).wait()

            gather(i + 1, rows1, sem_g1).start()

        @pl.when(valid(i + 1))
        def _():
            gather(i + 1, rows1, sem_g1).wait()
            writeback(i + 1, rows1, sem_w1).start()

        @pl.when(valid(i + 2))
        def _():
            writeback(i, rows0, sem_w0).wait()
            gather(i + 2, rows0, sem_g0).start()

        return carry

    lax.fori_loop(0, SLOTS_H // 2, step, 0)

    writeback(0, rows0, sem_w0).wait()
    writeback(0, rows1, sem_w1).wait()


_sc_gather_half = pl.kernel(
    _sc_gather_body,
    out_type=jax.ShapeDtypeStruct((HALF, IN_CH), jnp.float32),
    mesh=plsc.VectorSubcoreMesh(core_axis_name="c", subcore_axis_name="s"),
    scratch_types=[
        pltpu.VMEM((SLOTS_H, CHUNK), jnp.int32),
        pltpu.VMEM((CHUNK, IN_CH), jnp.float32),
        pltpu.VMEM((CHUNK, IN_CH), jnp.float32),
        pltpu.SemaphoreType.DMA,
        pltpu.SemaphoreType.DMA,
        pltpu.SemaphoreType.DMA,
        pltpu.SemaphoreType.DMA,
    ],
)


def _tc_fused_body(res_ref, gat_ref, wskip_ref, wproj_ref, bias_ref, out_ref):
    out_ref[...] = (
        jnp.dot(res_ref[...], wskip_ref[...], preferred_element_type=jnp.float32)
        + jnp.dot(gat_ref[...], wproj_ref[...], preferred_element_type=jnp.float32)
        + bias_ref[...]
    )


def _tc_fused_body2(res_ref, gat_ref, wskip_ref, wproj_ref, bias_ref, part_ref,
                    out_ref):
    del part_ref
    _tc_fused_body(res_ref, gat_ref, wskip_ref, wproj_ref, bias_ref, out_ref)


ROWS_BLK = 5000
GRID_H = HALF // ROWS_BLK

_W_SPECS = [
    pl.BlockSpec((SKIP_CH, OUT_CH), lambda i: (0, 0)),
    pl.BlockSpec((IN_CH, OUT_CH), lambda i: (0, 0)),
    pl.BlockSpec((1, OUT_CH), lambda i: (0, 0)),
]

_tc_first = pl.pallas_call(
    _tc_fused_body,
    grid=(GRID_H,),
    in_specs=[
        pl.BlockSpec((ROWS_BLK, SKIP_CH), lambda i: (i, 0)),
        pl.BlockSpec((ROWS_BLK, IN_CH), lambda i: (i, 0)),
        *_W_SPECS,
    ],
    out_specs=pl.BlockSpec((ROWS_BLK, OUT_CH), lambda i: (i, 0)),
    out_shape=jax.ShapeDtypeStruct((N_FINE, OUT_CH), jnp.float32),
)

_tc_second = pl.pallas_call(
    _tc_fused_body2,
    grid=(GRID_H,),
    in_specs=[
        pl.BlockSpec((ROWS_BLK, SKIP_CH), lambda i: (i + GRID_H, 0)),
        pl.BlockSpec((ROWS_BLK, IN_CH), lambda i: (i, 0)),
        *_W_SPECS,
        pl.BlockSpec(memory_space=pl.ANY),
    ],
    out_specs=pl.BlockSpec((ROWS_BLK, OUT_CH), lambda i: (i + GRID_H, 0)),
    out_shape=jax.ShapeDtypeStruct((N_FINE, OUT_CH), jnp.float32),
    input_output_aliases={5: 0},
)


def _half_idx(buffers_half):
    pad = SLOTS_H * NUM_WORKERS * CHUNK - HALF
    return jnp.pad(buffers_half, (0, pad)).reshape(SLOTS_H, NUM_WORKERS, CHUNK)


def kernel(residual, down, buffers, W_proj, b_proj, W_skip, b_skip):
    bias = (b_proj + b_skip).reshape(1, OUT_CH)
    g0 = _sc_gather_half(_half_idx(buffers[:HALF]), down)
    g1 = _sc_gather_half(_half_idx(buffers[HALF:]), down)
    part = _tc_first(residual, g0, W_skip, W_proj, bias)
    return _tc_second(residual, g1, W_skip, W_proj, bias, part)
